# Initial kernel scaffold; baseline (speedup 1.0000x reference)
import jax
import jax.numpy as jnp
from jax.experimental import pallas as pl


def kernel(token_ids, table, W, b, gamma, beta):
    raise NotImplementedError("write your pallas kernel here")


# speedup vs baseline: 31.5971x; 31.5971x over previous
import functools

import jax
import jax.numpy as jnp
from jax import lax
from jax.experimental import pallas as pl
from jax.experimental.pallas import tpu as pltpu
from jax.experimental.pallas import tpu_sc as plsc

V = 4096
D = 72
F = 256
L = 200

NC = 2
NS = 16
NW = NC * NS
QG = 16
VW = V // 4


def _sc_histogram(ids_hbm, cnt_hbm, ids_v, buf):
    wid = lax.axis_index("s") * NC + lax.axis_index("c")
    B = ids_hbm.shape[0] // L
    q_per_w = B // NW
    groups = q_per_w // QG
    base_q = wid * q_per_w

    lane = lax.iota(jnp.int32, 16)
    ones = jnp.ones((16,), jnp.int32)
    zeros = jnp.zeros((16,), jnp.int32)
    three = jnp.full((16,), 3, jnp.int32)
    lane_l = lane * L
    lane_w = lane * VW

    def group_body(g, _):
        q0 = base_q + g * QG
        pltpu.sync_copy(ids_hbm.at[pl.ds(q0 * L, QG * L)], ids_v)

        def zrow(i, _):
            buf[pl.ds(i * 16, 16)] = zeros
            return 0

        lax.fori_loop(0, QG * VW // 16, zrow, 0)

        def tok_body(l, _):
            tok = plsc.load_gather(ids_v, [lane_l + l])
            word = lax.shift_right_logical(tok, 2)
            shift = lax.shift_left(jnp.bitwise_and(tok, three), 3)
            addend = lax.shift_left(ones, shift)
            plsc.addupdate_scatter(buf, [lane_w + word], addend)
            return 0

        lax.fori_loop(0, L, tok_body, 0)

        pltpu.sync_copy(buf, cnt_hbm.at[pl.ds(q0 * VW, QG * VW)])
        return 0

    lax.fori_loop(0, groups, group_body, 0)


def _tc_project(cnt_ref, tab_ref, w_ref, b_ref, g_ref, be_ref, out_ref):
    cnt = cnt_ref[...]
    planes = [
        jnp.bitwise_and(lax.shift_right_logical(cnt, 8 * k), 255)
        for k in range(4)
    ]
    cb = jnp.concatenate(planes, axis=1).astype(jnp.bfloat16)
    pooled = lax.dot_general(
        cb, tab_ref[...], (((1,), (0,)), ((), ())),
        preferred_element_type=jnp.float32) * (1.0 / L)
    h = lax.dot_general(
        pooled, w_ref[...], (((1,), (0,)), ((), ())),
        preferred_element_type=jnp.float32) + b_ref[...]
    mu = jnp.mean(h, axis=-1, keepdims=True)
    d = h - mu
    var = jnp.mean(d * d, axis=-1, keepdims=True)
    hn = d * lax.rsqrt(var + 1e-5)
    out_ref[...] = jnp.maximum(hn * g_ref[...] + be_ref[...], 0.0)


def kernel(token_ids, table, W, b, gamma, beta):
    B = token_ids.shape[0]

    mesh = plsc.VectorSubcoreMesh(core_axis_name="c", subcore_axis_name="s")
    counts = pl.kernel(
        _sc_histogram,
        out_type=jax.ShapeDtypeStruct((B * VW,), jnp.int32),
        mesh=mesh,
        scratch_types=[
            pltpu.VMEM((QG * L,), jnp.int32),
            pltpu.VMEM((QG * VW,), jnp.int32),
        ],
        compiler_params=pltpu.CompilerParams(needs_layout_passes=False),
    )(token_ids.reshape(-1)).reshape(B, VW)

    table_r = jnp.concatenate(
        [table[0::4], table[1::4], table[2::4], table[3::4]], axis=0
    ).astype(jnp.bfloat16)

    BB = 512
    grid = B // BB
    out = pl.pallas_call(
        _tc_project,
        grid=(grid,),
        in_specs=[
            pl.BlockSpec((BB, VW), lambda i: (i, 0)),
            pl.BlockSpec((V, D), lambda i: (0, 0)),
            pl.BlockSpec((D, F), lambda i: (0, 0)),
            pl.BlockSpec((1, F), lambda i: (0, 0)),
            pl.BlockSpec((1, F), lambda i: (0, 0)),
            pl.BlockSpec((1, F), lambda i: (0, 0)),
        ],
        out_specs=pl.BlockSpec((BB, F), lambda i: (i, 0)),
        out_shape=jax.ShapeDtypeStruct((B, F), jnp.float32),
    )(counts, table_r, W, b.reshape(1, F), gamma.reshape(1, F),
      beta.reshape(1, F))
    return out
